# Initial kernel scaffold; baseline (speedup 1.0000x reference)
import jax
import jax.numpy as jnp
from jax.experimental import pallas as pl


def kernel(x, pos_embedding):
    raise NotImplementedError("write your pallas kernel here")


# speedup vs baseline: 1.0916x; 1.0916x over previous
import jax
import jax.numpy as jnp
from jax.experimental import pallas as pl


_BS = 512


def _add_kernel(x_ref, pe_ref, o_ref):
    o_ref[...] = x_ref[...] + pe_ref[...]


def kernel(x, pos_embedding):
    batch, seq_len, d = x.shape
    n_seq = seq_len // _BS
    return pl.pallas_call(
        _add_kernel,
        grid=(n_seq, batch),
        in_specs=[
            pl.BlockSpec((1, _BS, d), lambda s, b: (b, s, 0)),
            pl.BlockSpec((_BS, d), lambda s, b: (s, 0)),
        ],
        out_specs=pl.BlockSpec((1, _BS, d), lambda s, b: (b, s, 0)),
        out_shape=jax.ShapeDtypeStruct((batch, seq_len, d), x.dtype),
    )(x, pos_embedding)


# speedup vs baseline: 1.2245x; 1.1218x over previous
import jax
import jax.numpy as jnp
from jax.experimental import pallas as pl


_BS = 1024


def _add_kernel(x_ref, pe_ref, o_ref):
    o_ref[...] = x_ref[...] + pe_ref[...]


def kernel(x, pos_embedding):
    batch, seq_len, d = x.shape
    n_seq = seq_len // _BS
    return pl.pallas_call(
        _add_kernel,
        grid=(n_seq, batch),
        in_specs=[
            pl.BlockSpec((1, _BS, d), lambda s, b: (b, s, 0)),
            pl.BlockSpec((_BS, d), lambda s, b: (s, 0)),
        ],
        out_specs=pl.BlockSpec((1, _BS, d), lambda s, b: (b, s, 0)),
        out_shape=jax.ShapeDtypeStruct((batch, seq_len, d), x.dtype),
    )(x, pos_embedding)


# speedup vs baseline: 1.2947x; 1.0573x over previous
import jax
import jax.numpy as jnp
from jax.experimental import pallas as pl


_BS = 2048


def _add_kernel(x_ref, pe_ref, o_ref):
    o_ref[...] = x_ref[...] + pe_ref[...]


def kernel(x, pos_embedding):
    batch, seq_len, d = x.shape
    n_seq = seq_len // _BS
    return pl.pallas_call(
        _add_kernel,
        grid=(n_seq, batch),
        in_specs=[
            pl.BlockSpec((1, _BS, d), lambda s, b: (b, s, 0)),
            pl.BlockSpec((_BS, d), lambda s, b: (s, 0)),
        ],
        out_specs=pl.BlockSpec((1, _BS, d), lambda s, b: (b, s, 0)),
        out_shape=jax.ShapeDtypeStruct((batch, seq_len, d), x.dtype),
    )(x, pos_embedding)


# speedup vs baseline: 1.2997x; 1.0038x over previous
import jax
import jax.numpy as jnp
from jax.experimental import pallas as pl
from jax.experimental.pallas import tpu as pltpu


_BS = 2048


def _add_kernel(x_ref, pe_ref, o_ref):
    o_ref[...] = x_ref[...] + pe_ref[...]


def kernel(x, pos_embedding):
    batch, seq_len, d = x.shape
    n_seq = seq_len // _BS
    return pl.pallas_call(
        _add_kernel,
        grid=(n_seq, batch),
        in_specs=[
            pl.BlockSpec((1, _BS, d), lambda s, b: (b, s, 0)),
            pl.BlockSpec((_BS, d), lambda s, b: (s, 0)),
        ],
        out_specs=pl.BlockSpec((1, _BS, d), lambda s, b: (b, s, 0)),
        out_shape=jax.ShapeDtypeStruct((batch, seq_len, d), x.dtype),
        compiler_params=pltpu.CompilerParams(
            dimension_semantics=("parallel", "parallel"),
        ),
    )(x, pos_embedding)
